# Initial kernel scaffold; baseline (speedup 1.0000x reference)
import jax
import jax.numpy as jnp
from jax.experimental import pallas as pl


def kernel(vertices, reverse_map):
    raise NotImplementedError("write your pallas kernel here")


# speedup vs baseline: 1.3956x; 1.3956x over previous
import functools

import jax
import jax.numpy as jnp
from jax import lax
from jax.experimental import pallas as pl
from jax.experimental.pallas import tpu as pltpu
from jax.experimental.pallas import tpu_sc as plsc

_NC = 2
_NS = 16
_NW = _NC * _NS
_CH = 125


def _sc_gather(table, idx3):
    nw, nch, ch = idx3.shape
    rows_total = nw * nch * ch
    d = table.shape[1]
    per_w = nch * ch
    mesh = plsc.VectorSubcoreMesh(core_axis_name="c", subcore_axis_name="s")

    @functools.partial(
        pl.kernel,
        mesh=mesh,
        out_type=jax.ShapeDtypeStruct((rows_total, d), jnp.float32),
        scratch_types=[
            pltpu.VMEM((nch, ch), jnp.int32),
            pltpu.VMEM((ch, d), jnp.float32),
            pltpu.SemaphoreType.DMA,
        ],
        compiler_params=pltpu.CompilerParams(use_tc_tiling_on_sc=False),
    )
    def gather_kernel(table_hbm, idx_hbm, out_hbm, idx_v, rows_v, sem):
        wid = lax.axis_index("s") * _NC + lax.axis_index("c")
        pltpu.sync_copy(idx_hbm.at[wid], idx_v)
        base = wid * per_w

        def step(j, carry):
            pltpu.async_copy(table_hbm.at[idx_v.at[j]], rows_v, sem).wait()
            pltpu.sync_copy(rows_v, out_hbm.at[pl.ds(base + j * ch, ch)])
            return carry

        lax.fori_loop(0, nch, step, 0)

    return gather_kernel(table, idx3)


def kernel(vertices, reverse_map):
    b, n, d = vertices.shape
    table = vertices.reshape(b * n, d)
    offs = (jnp.arange(b, dtype=jnp.int32) * n)[:, None]
    flat_idx = (reverse_map.astype(jnp.int32) + offs).reshape(-1)
    nch = (b * n) // (_NW * _CH)
    idx3 = flat_idx.reshape(_NW, nch, _CH)
    out = _sc_gather(table, idx3)
    return out.reshape(b, n, d)


# speedup vs baseline: 1.5876x; 1.1376x over previous
import functools

import jax
import jax.numpy as jnp
from jax import lax
from jax.experimental import pallas as pl
from jax.experimental.pallas import tpu as pltpu
from jax.experimental.pallas import tpu_sc as plsc

_NC = 2
_NS = 16
_NW = _NC * _NS
_CH = 125


def _sc_gather(table, idx3):
    nw, nch, ch = idx3.shape
    rows_total = nw * nch * ch
    d = table.shape[1]
    per_w = nch * ch
    mesh = plsc.VectorSubcoreMesh(core_axis_name="c", subcore_axis_name="s")

    @functools.partial(
        pl.kernel,
        mesh=mesh,
        out_type=jax.ShapeDtypeStruct((rows_total, d), jnp.float32),
        scratch_types=[
            pltpu.VMEM((nch, ch), jnp.int32),
            pltpu.VMEM((ch, d), jnp.float32),
            pltpu.VMEM((ch, d), jnp.float32),
            pltpu.SemaphoreType.DMA,
            pltpu.SemaphoreType.DMA,
        ],
        compiler_params=pltpu.CompilerParams(use_tc_tiling_on_sc=False),
    )
    def gather_kernel(table_hbm, idx_hbm, out_hbm, idx_v, rows_a, rows_b,
                      sem_a, sem_b):
        wid = lax.axis_index("s") * _NC + lax.axis_index("c")
        pltpu.sync_copy(idx_hbm.at[wid], idx_v)
        base = wid * per_w

        pltpu.async_copy(table_hbm.at[idx_v.at[0]], rows_a, sem_a)

        def step(i, carry):
            j = 2 * i
            pltpu.make_async_copy(
                table_hbm.at[pl.ds(0, ch)], rows_a, sem_a).wait()
            pltpu.async_copy(table_hbm.at[idx_v.at[j + 1]], rows_b, sem_b)
            pltpu.sync_copy(rows_a, out_hbm.at[pl.ds(base + j * ch, ch)])
            pltpu.make_async_copy(
                table_hbm.at[pl.ds(0, ch)], rows_b, sem_b).wait()

            @pl.when(j + 2 < nch)
            def _():
                pltpu.async_copy(table_hbm.at[idx_v.at[j + 2]], rows_a, sem_a)

            pltpu.sync_copy(rows_b, out_hbm.at[pl.ds(base + (j + 1) * ch, ch)])
            return carry

        lax.fori_loop(0, nch // 2, step, 0)

    return gather_kernel(table, idx3)


def kernel(vertices, reverse_map):
    b, n, d = vertices.shape
    table = vertices.reshape(b * n, d)
    offs = (jnp.arange(b, dtype=jnp.int32) * n)[:, None]
    flat_idx = (reverse_map.astype(jnp.int32) + offs).reshape(-1)
    nch = (b * n) // (_NW * _CH)
    idx3 = flat_idx.reshape(_NW, nch, _CH)
    out = _sc_gather(table, idx3)
    return out.reshape(b, n, d)


# speedup vs baseline: 1.9514x; 1.2292x over previous
import functools

import jax
import jax.numpy as jnp
from jax import lax
from jax.experimental import pallas as pl
from jax.experimental.pallas import tpu as pltpu
from jax.experimental.pallas import tpu_sc as plsc

_NC = 2
_NS = 16
_NW = _NC * _NS
_CH = 125


def _sc_gather(table, idx3):
    nw, nch, ch = idx3.shape
    assert nch % 4 == 2, "pipeline peel/epilogue assumes nch ≡ 2 (mod 4)"
    rows_total = nw * nch * ch
    d = table.shape[1]
    per_w = nch * ch
    mesh = plsc.VectorSubcoreMesh(core_axis_name="c", subcore_axis_name="s")

    @functools.partial(
        pl.kernel,
        mesh=mesh,
        out_type=jax.ShapeDtypeStruct((rows_total, d), jnp.float32),
        scratch_types=[
            pltpu.VMEM((nch, ch), jnp.int32),
            [pltpu.VMEM((ch, d), jnp.float32) for _ in range(4)],
            [pltpu.SemaphoreType.DMA for _ in range(4)],
            [pltpu.SemaphoreType.DMA for _ in range(4)],
        ],
        compiler_params=pltpu.CompilerParams(use_tc_tiling_on_sc=False),
    )
    def gather_kernel(table_hbm, idx_hbm, out_hbm, idx_v, rows, gsem, ssem):
        wid = lax.axis_index("s") * _NC + lax.axis_index("c")
        pltpu.sync_copy(idx_hbm.at[wid], idx_v)
        base = wid * per_w

        def fire_gather(j, b):
            pltpu.async_copy(table_hbm.at[idx_v.at[j]], rows[b], gsem[b])

        def wait_gather(b):
            pltpu.make_async_copy(
                table_hbm.at[pl.ds(0, ch)], rows[b], gsem[b]).wait()

        def fire_store(j, b):
            pltpu.async_copy(
                rows[b], out_hbm.at[pl.ds(base + j * ch, ch)], ssem[b])

        def wait_store(b):
            pltpu.make_async_copy(
                rows[b], out_hbm.at[pl.ds(0, ch)], ssem[b]).wait()

        fire_gather(0, 0)
        fire_gather(1, 1)
        for b in range(4):
            wait_gather(b)
            fire_store(b, b)
            if b >= 2:
                wait_store((b + 2) % 4)
            fire_gather(b + 2, (b + 2) % 4)

        def step(k, carry):
            for b in range(4):
                j = 4 * k + b
                wait_gather(b)
                fire_store(j, b)
                t = (b + 2) % 4
                wait_store(t)
                fire_gather(j + 2, t)
            return carry

        lax.fori_loop(1, nch // 4, step, 0)

        for b in range(2):
            wait_gather(b)
            fire_store(nch - 2 + b, b)
        for b in (2, 3, 0, 1):
            wait_store(b)

    return gather_kernel(table, idx3)


def kernel(vertices, reverse_map):
    b, n, d = vertices.shape
    table = vertices.reshape(b * n, d)
    offs = (jnp.arange(b, dtype=jnp.int32) * n)[:, None]
    flat_idx = (reverse_map.astype(jnp.int32) + offs).reshape(-1)
    nch = (b * n) // (_NW * _CH)
    idx3 = flat_idx.reshape(_NW, nch, _CH)
    out = _sc_gather(table, idx3)
    return out.reshape(b, n, d)


# speedup vs baseline: 1.9594x; 1.0041x over previous
import functools

import jax
import jax.numpy as jnp
from jax import lax
from jax.experimental import pallas as pl
from jax.experimental.pallas import tpu as pltpu
from jax.experimental.pallas import tpu_sc as plsc

_NC = 2
_NS = 16
_NW = _NC * _NS
_CH = 125


def _sc_gather(table, idx3):
    nw, nch, ch = idx3.shape
    assert nch % 4 == 2, "pipeline peel/epilogue assumes nch ≡ 2 (mod 4)"
    rows_total = nw * nch * ch
    nb, n, d = table.shape
    per_w = nch * ch
    w_per_b = nw // nb
    assert w_per_b * nb == nw and per_w * w_per_b == n
    mesh = plsc.VectorSubcoreMesh(core_axis_name="c", subcore_axis_name="s")

    @functools.partial(
        pl.kernel,
        mesh=mesh,
        out_type=jax.ShapeDtypeStruct((rows_total, d), jnp.float32),
        scratch_types=[
            pltpu.VMEM((nch, ch), jnp.int32),
            [pltpu.VMEM((ch, d), jnp.float32) for _ in range(4)],
            [pltpu.SemaphoreType.DMA for _ in range(4)],
            [pltpu.SemaphoreType.DMA for _ in range(4)],
        ],
        compiler_params=pltpu.CompilerParams(use_tc_tiling_on_sc=False),
    )
    def gather_kernel(table_hbm, idx_hbm, out_hbm, idx_v, rows, gsem, ssem):
        wid = lax.axis_index("s") * _NC + lax.axis_index("c")
        pltpu.sync_copy(idx_hbm.at[wid], idx_v)
        base = wid * per_w
        batch = wid // w_per_b

        def fire_gather(j, b):
            pltpu.async_copy(
                table_hbm.at[batch].at[idx_v.at[j]], rows[b], gsem[b])

        def wait_gather(b):
            pltpu.make_async_copy(
                table_hbm.at[0].at[pl.ds(0, ch)], rows[b], gsem[b]).wait()

        def fire_store(j, b):
            pltpu.async_copy(
                rows[b], out_hbm.at[pl.ds(base + j * ch, ch)], ssem[b])

        def wait_store(b):
            pltpu.make_async_copy(
                rows[b], out_hbm.at[pl.ds(0, ch)], ssem[b]).wait()

        fire_gather(0, 0)
        fire_gather(1, 1)
        for b in range(4):
            wait_gather(b)
            fire_store(b, b)
            if b >= 2:
                wait_store((b + 2) % 4)
            fire_gather(b + 2, (b + 2) % 4)

        def step(k, carry):
            for b in range(4):
                j = 4 * k + b
                wait_gather(b)
                fire_store(j, b)
                t = (b + 2) % 4
                wait_store(t)
                fire_gather(j + 2, t)
            return carry

        lax.fori_loop(1, nch // 4, step, 0)

        for b in range(2):
            wait_gather(b)
            fire_store(nch - 2 + b, b)
        for b in (2, 3, 0, 1):
            wait_store(b)

    return gather_kernel(table, idx3)


def kernel(vertices, reverse_map):
    b, n, d = vertices.shape
    nch = (b * n) // (_NW * _CH)
    idx3 = reverse_map.astype(jnp.int32).reshape(_NW, nch, _CH)
    out = _sc_gather(vertices, idx3)
    return out.reshape(b, n, d)


# speedup vs baseline: 1.9835x; 1.0123x over previous
import functools

import jax
import jax.numpy as jnp
from jax import lax
from jax.experimental import pallas as pl
from jax.experimental.pallas import tpu as pltpu
from jax.experimental.pallas import tpu_sc as plsc

_NC = 2
_NS = 16
_NW = _NC * _NS
_CH = 125


def _sc_gather(table, idx3):
    nw, nch, ch = idx3.shape
    ring, ahead = 6, 3
    full_blocks = nch // ring
    rem = nch % ring
    assert full_blocks >= 3 and 0 < rem < ahead
    rows_total = nw * nch * ch
    nb, n, d = table.shape
    per_w = nch * ch
    w_per_b = nw // nb
    assert w_per_b * nb == nw and per_w * w_per_b == n
    mesh = plsc.VectorSubcoreMesh(core_axis_name="c", subcore_axis_name="s")

    @functools.partial(
        pl.kernel,
        mesh=mesh,
        out_type=jax.ShapeDtypeStruct((rows_total, d), jnp.float32),
        scratch_types=[
            pltpu.VMEM((nch, ch), jnp.int32),
            [pltpu.VMEM((ch, d), jnp.float32) for _ in range(ring)],
            [pltpu.SemaphoreType.DMA for _ in range(ring)],
            [pltpu.SemaphoreType.DMA for _ in range(ring)],
        ],
        compiler_params=pltpu.CompilerParams(use_tc_tiling_on_sc=False),
    )
    def gather_kernel(table_hbm, idx_hbm, out_hbm, idx_v, rows, gsem, ssem):
        wid = lax.axis_index("s") * _NC + lax.axis_index("c")
        pltpu.sync_copy(idx_hbm.at[wid], idx_v)
        base = wid * per_w
        batch = wid // w_per_b

        def fire_gather(j, b):
            pltpu.async_copy(
                table_hbm.at[batch].at[idx_v.at[j]], rows[b], gsem[b])

        def wait_gather(b):
            pltpu.make_async_copy(
                table_hbm.at[0].at[pl.ds(0, ch)], rows[b], gsem[b]).wait()

        def fire_store(j, b):
            pltpu.async_copy(
                rows[b], out_hbm.at[pl.ds(base + j * ch, ch)], ssem[b])

        def wait_store(b):
            pltpu.make_async_copy(
                rows[b], out_hbm.at[pl.ds(0, ch)], ssem[b]).wait()

        for f in range(ahead):
            fire_gather(f, f)

        for b in range(ring):
            wait_gather(b)
            fire_store(b, b)
            t = (b + ahead) % ring
            if b >= ahead:
                wait_store(t)
            fire_gather(b + ahead, t)

        def step(k, carry):
            for b in range(ring):
                j = ring * k + b
                wait_gather(b)
                fire_store(j, b)
                t = (b + ahead) % ring
                wait_store(t)
                fire_gather(j + ahead, t)
            return carry

        lax.fori_loop(1, full_blocks - 1, step, 0)

        for b in range(ring):
            j = ring * (full_blocks - 1) + b
            wait_gather(b)
            fire_store(j, b)
            if j + ahead < nch:
                t = (b + ahead) % ring
                wait_store(t)
                fire_gather(j + ahead, t)

        for r in range(rem):
            j = ring * full_blocks + r
            wait_gather(j % ring)
            fire_store(j, j % ring)
        for j in range(nch - ring, nch):
            wait_store(j % ring)

    return gather_kernel(table, idx3)


def kernel(vertices, reverse_map):
    b, n, d = vertices.shape
    nch = (b * n) // (_NW * _CH)
    idx3 = reverse_map.astype(jnp.int32).reshape(_NW, nch, _CH)
    out = _sc_gather(vertices, idx3)
    return out.reshape(b, n, d)


# speedup vs baseline: 2.0110x; 1.0139x over previous
import functools

import jax
import jax.numpy as jnp
from jax import lax
from jax.experimental import pallas as pl
from jax.experimental.pallas import tpu as pltpu
from jax.experimental.pallas import tpu_sc as plsc

_NC = 2
_NS = 16
_NW = _NC * _NS
_CH = 125


def _sc_gather(table, idx3):
    nw, nch, ch = idx3.shape
    ring, ahead = 7, 4
    full_blocks = nch // ring
    rem = nch % ring
    assert full_blocks >= 3 and 0 < rem < ahead
    rows_total = nw * nch * ch
    nb, n, d = table.shape
    per_w = nch * ch
    w_per_b = nw // nb
    assert w_per_b * nb == nw and per_w * w_per_b == n
    mesh = plsc.VectorSubcoreMesh(core_axis_name="c", subcore_axis_name="s")

    @functools.partial(
        pl.kernel,
        mesh=mesh,
        out_type=jax.ShapeDtypeStruct((rows_total, d), jnp.float32),
        scratch_types=[
            pltpu.VMEM((nch, ch), jnp.int32),
            [pltpu.VMEM((ch, d), jnp.float32) for _ in range(ring)],
            [pltpu.SemaphoreType.DMA for _ in range(ring)],
            [pltpu.SemaphoreType.DMA for _ in range(ring)],
        ],
        compiler_params=pltpu.CompilerParams(use_tc_tiling_on_sc=False),
    )
    def gather_kernel(table_hbm, idx_hbm, out_hbm, idx_v, rows, gsem, ssem):
        wid = lax.axis_index("s") * _NC + lax.axis_index("c")
        pltpu.sync_copy(idx_hbm.at[wid], idx_v)
        base = wid * per_w
        batch = wid // w_per_b

        def fire_gather(j, b):
            pltpu.async_copy(
                table_hbm.at[batch].at[idx_v.at[j]], rows[b], gsem[b])

        def wait_gather(b):
            pltpu.make_async_copy(
                table_hbm.at[0].at[pl.ds(0, ch)], rows[b], gsem[b]).wait()

        def fire_store(j, b):
            pltpu.async_copy(
                rows[b], out_hbm.at[pl.ds(base + j * ch, ch)], ssem[b])

        def wait_store(b):
            pltpu.make_async_copy(
                rows[b], out_hbm.at[pl.ds(0, ch)], ssem[b]).wait()

        for f in range(ahead):
            fire_gather(f, f)

        for b in range(ring):
            wait_gather(b)
            fire_store(b, b)
            t = (b + ahead) % ring
            if b >= ahead:
                wait_store(t)
            fire_gather(b + ahead, t)

        def step(k, carry):
            for b in range(ring):
                j = ring * k + b
                wait_gather(b)
                fire_store(j, b)
                t = (b + ahead) % ring
                wait_store(t)
                fire_gather(j + ahead, t)
            return carry

        lax.fori_loop(1, full_blocks - 1, step, 0)

        for b in range(ring):
            j = ring * (full_blocks - 1) + b
            wait_gather(b)
            fire_store(j, b)
            if j + ahead < nch:
                t = (b + ahead) % ring
                wait_store(t)
                fire_gather(j + ahead, t)

        for r in range(rem):
            j = ring * full_blocks + r
            wait_gather(j % ring)
            fire_store(j, j % ring)
        for j in range(nch - ring, nch):
            wait_store(j % ring)

    return gather_kernel(table, idx3)


def kernel(vertices, reverse_map):
    b, n, d = vertices.shape
    nch = (b * n) // (_NW * _CH)
    idx3 = reverse_map.astype(jnp.int32).reshape(_NW, nch, _CH)
    out = _sc_gather(vertices, idx3)
    return out.reshape(b, n, d)
